# Initial kernel scaffold; baseline (speedup 1.0000x reference)
import jax
import jax.numpy as jnp
from jax.experimental import pallas as pl


def kernel(x, W_enc, W_dec, b_dec):
    raise NotImplementedError("write your pallas kernel here")


# speedup vs baseline: 21.1777x; 21.1777x over previous
import functools

import jax
import jax.numpy as jnp
from jax import lax
from jax.experimental import pallas as pl
from jax.experimental.pallas import tpu as pltpu
from jax.experimental.pallas import tpu_sc as plsc

_B = 2048
_D_IN = 1024
_D_SAE = 8192
_K = _B * 32
_N = _B * _D_SAE

_NC = 2
_NS = 16
_NW = _NC * _NS
_PER_W = _N // _NW
_CHUNK = 16384
_NCHUNK = _PER_W // _CHUNK


def _encode_body(x_ref, w_ref, o_ref):
    acc = jnp.dot(x_ref[...], w_ref[...], preferred_element_type=jnp.float32)
    o_ref[...] = jnp.maximum(acc, 0.0)


def _encode(xc, w_enc):
    bn = 1024
    return pl.pallas_call(
        _encode_body,
        grid=(_D_SAE // bn,),
        in_specs=[
            pl.BlockSpec((_B, _D_IN), lambda n: (0, 0)),
            pl.BlockSpec((_D_IN, bn), lambda n: (0, n)),
        ],
        out_specs=pl.BlockSpec((_B, bn), lambda n: (0, n)),
        out_shape=jax.ShapeDtypeStruct((_B, _D_SAE), jnp.float32),
    )(xc, w_enc)


def _sc_hist1(flat):
    mesh = plsc.VectorSubcoreMesh(core_axis_name="c", subcore_axis_name="s")

    @functools.partial(
        pl.kernel,
        mesh=mesh,
        out_type=jax.ShapeDtypeStruct((_NW, 4096), jnp.int32),
        compiler_params=pltpu.CompilerParams(needs_layout_passes=False),
        scratch_types=[
            pltpu.VMEM((_CHUNK,), jnp.int32),
            pltpu.VMEM((16 * 4096,), jnp.int32),
            pltpu.VMEM((4096,), jnp.int32),
        ],
    )
    def k(flat_hbm, out_hbm, buf, hist, lred):
        sid = lax.axis_index("s")
        cid = lax.axis_index("c")
        wid = sid * _NC + cid
        lanes = lax.iota(jnp.int32, 16)
        lanes4k = lanes * 4096
        zeros16 = jnp.zeros((16,), jnp.int32)
        ones16 = jnp.ones((16,), jnp.int32)

        def zh(j, _):
            hist[pl.ds(j * 16, 16)] = zeros16
            return 0

        lax.fori_loop(0, 4096, zh, 0)

        base = wid * _PER_W

        def chunk_body(g, _):
            pltpu.sync_copy(flat_hbm.at[pl.ds(base + g * _CHUNK, _CHUNK)], buf)

            def inner(i, _):
                bits = jnp.maximum(buf[pl.ds(i * 16, 16)], 0)
                bkt = lax.shift_right_logical(bits, 19)
                plsc.addupdate_scatter(hist, [bkt + lanes4k], ones16)
                return 0

            lax.fori_loop(0, _CHUNK // 16, inner, 0)
            return 0

        lax.fori_loop(0, _NCHUNK, chunk_body, 0)

        def red(j, _):
            acc = zeros16
            for lane in range(16):
                acc = acc + hist[pl.ds(lane * 4096 + j * 16, 16)]
            lred[pl.ds(j * 16, 16)] = acc
            return 0

        lax.fori_loop(0, 256, red, 0)
        pltpu.sync_copy(lred, out_hbm.at[wid])

    return k(flat)


def _scan1_body(h1_ref, o_ref):
    h = jnp.sum(h1_ref[...].astype(jnp.float32), axis=0)
    h2d = h.reshape(32, 128)
    li = lax.broadcasted_iota(jnp.int32, (128, 128), 0)
    lj = lax.broadcasted_iota(jnp.int32, (128, 128), 1)
    lmat = (li <= lj).astype(jnp.float32)
    rowcum = jnp.dot(h2d, lmat, preferred_element_type=jnp.float32,
                     precision=lax.Precision.HIGHEST)
    si = lax.broadcasted_iota(jnp.int32, (32, 32), 0)
    sj = lax.broadcasted_iota(jnp.int32, (32, 32), 1)
    smat = (sj < si).astype(jnp.float32)
    rowtot = rowcum[:, 127:128]
    rowoff = jnp.dot(smat, rowtot, preferred_element_type=jnp.float32,
                     precision=lax.Precision.HIGHEST)
    prefexc = rowcum + rowoff - h2d
    total = jnp.sum(h2d)
    cond = (prefexc <= (total - float(_K))).astype(jnp.float32)
    b1 = jnp.sum(cond).astype(jnp.int32) - 1
    b1f = jnp.sum(cond) - 1.0
    flat_idx = (lax.broadcasted_iota(jnp.int32, (32, 128), 0) * 128 +
                lax.broadcasted_iota(jnp.int32, (32, 128), 1)).astype(jnp.float32)
    sel = (flat_idx == b1f).astype(jnp.float32)
    prefinc_b1 = jnp.sum(sel * (prefexc + h2d))
    r = jnp.float32(_K) - (total - prefinc_b1)
    o_ref[0, :] = jnp.broadcast_to(b1, (128,))
    o_ref[1, :] = jnp.broadcast_to(r.astype(jnp.int32), (128,))


def _scan1(h1):
    return pl.pallas_call(
        _scan1_body,
        in_specs=[pl.BlockSpec((_NW, 4096), lambda: (0, 0))],
        out_specs=pl.BlockSpec((2, 128), lambda: (0, 0)),
        out_shape=jax.ShapeDtypeStruct((2, 128), jnp.int32),
    )(h1)


def _sc_hist2(flat, b1vec):
    mesh = plsc.VectorSubcoreMesh(core_axis_name="c", subcore_axis_name="s")

    @functools.partial(
        pl.kernel,
        mesh=mesh,
        out_type=jax.ShapeDtypeStruct((_NW, 65536), jnp.int32),
        compiler_params=pltpu.CompilerParams(needs_layout_passes=False),
        scratch_types=[
            pltpu.VMEM((_CHUNK,), jnp.int32),
            pltpu.VMEM((65536,), jnp.int32),
            pltpu.VMEM((16,), jnp.int32),
        ],
    )
    def k(flat_hbm, b1_hbm, out_hbm, buf, hist, b1buf):
        sid = lax.axis_index("s")
        cid = lax.axis_index("c")
        wid = sid * _NC + cid
        zeros16 = jnp.zeros((16,), jnp.int32)
        ones16 = jnp.ones((16,), jnp.int32)

        pltpu.sync_copy(b1_hbm, b1buf)
        b1v = b1buf[pl.ds(0, 16)]

        def zh(j, _):
            hist[pl.ds(j * 16, 16)] = zeros16
            return 0

        lax.fori_loop(0, 4096, zh, 0)

        base = wid * _PER_W

        def chunk_body(g, _):
            pltpu.sync_copy(flat_hbm.at[pl.ds(base + g * _CHUNK, _CHUNK)], buf)

            def inner(i, _):
                bits = jnp.maximum(buf[pl.ds(i * 16, 16)], 0)
                bkt = lax.shift_right_logical(bits, 19)
                m = bkt == b1v
                sub = lax.shift_right_logical(bits, 3) & 0xFFFF
                plsc.addupdate_scatter(hist, [sub], ones16, mask=m)
                return 0

            lax.fori_loop(0, _CHUNK // 16, inner, 0)
            return 0

        lax.fori_loop(0, _NCHUNK, chunk_body, 0)
        pltpu.sync_copy(hist, out_hbm.at[wid])

    return k(flat, b1vec)


def _decode_body(h2_ref, binfo_ref, dense_ref, w_ref, bdec_ref,
                 recon_ref, sparse_ref, t_ref):
    j = pl.program_id(0)

    @pl.when(j == 0)
    def _():
        h = jnp.sum(h2_ref[...].astype(jnp.float32), axis=0)
        li = lax.broadcasted_iota(jnp.int32, (128, 128), 0)
        lj = lax.broadcasted_iota(jnp.int32, (128, 128), 1)
        lmat = (li <= lj).astype(jnp.float32)
        rowcum = jnp.dot(h, lmat, preferred_element_type=jnp.float32,
                         precision=lax.Precision.HIGHEST)
        si = lax.broadcasted_iota(jnp.int32, (512, 512), 0)
        sj = lax.broadcasted_iota(jnp.int32, (512, 512), 1)
        smat = (sj < si).astype(jnp.float32)
        rowtot = rowcum[:, 127:128]
        rowoff = jnp.dot(smat, rowtot, preferred_element_type=jnp.float32,
                         precision=lax.Precision.HIGHEST)
        prefexc = rowcum + rowoff - h
        total = jnp.sum(h)
        r = binfo_ref[0, 1].astype(jnp.float32)
        cond = (prefexc <= (total - r)).astype(jnp.float32)
        b2 = jnp.sum(cond).astype(jnp.int32) - 1
        b1 = binfo_ref[0, 0]
        t_ref[0, 0] = (b1 << 19) | (b2 << 3)

    tbits = t_ref[0, 0]
    d = dense_ref[...]
    dbits = lax.bitcast_convert_type(d, jnp.int32)
    s = jnp.where(dbits >= tbits, d, 0.0)
    sparse_ref[...] = s

    @pl.when(j == 0)
    def _():
        recon_ref[...] = jnp.broadcast_to(bdec_ref[...], recon_ref.shape)

    recon_ref[...] += jnp.dot(s, w_ref[...], preferred_element_type=jnp.float32)


def _decode(dense, w_dec, bdec2d, h2, binfo):
    bk = 512
    return pl.pallas_call(
        _decode_body,
        grid=(_D_SAE // bk,),
        in_specs=[
            pl.BlockSpec((_NW, 512, 128), lambda n: (0, 0, 0)),
            pl.BlockSpec(memory_space=pltpu.SMEM),
            pl.BlockSpec((_B, bk), lambda n: (0, n)),
            pl.BlockSpec((bk, _D_IN), lambda n: (n, 0)),
            pl.BlockSpec((1, _D_IN), lambda n: (0, 0)),
        ],
        out_specs=[
            pl.BlockSpec((_B, _D_IN), lambda n: (0, 0)),
            pl.BlockSpec((_B, bk), lambda n: (0, n)),
        ],
        out_shape=[
            jax.ShapeDtypeStruct((_B, _D_IN), jnp.float32),
            jax.ShapeDtypeStruct((_B, _D_SAE), jnp.float32),
        ],
        scratch_shapes=[pltpu.SMEM((1, 1), jnp.int32)],
    )(h2, binfo, dense, w_dec, bdec2d)


def kernel(x, W_enc, W_dec, b_dec):
    xc = x - b_dec
    dense = _encode(xc, W_enc)
    flat = lax.bitcast_convert_type(dense, jnp.int32).reshape(-1)
    h1 = _sc_hist1(flat)
    info = _scan1(h1)
    b1vec = info[0, :16]
    binfo = info[:, :1].reshape(1, 2)
    h2 = _sc_hist2(flat, b1vec)
    x_recon, sparse = _decode(dense, W_dec, b_dec.reshape(1, _D_IN),
                              h2.reshape(_NW, 512, 128), binfo)
    return x_recon, sparse


# speedup vs baseline: 23.6881x; 1.1185x over previous
import functools

import jax
import jax.numpy as jnp
from jax import lax
from jax.experimental import pallas as pl
from jax.experimental.pallas import tpu as pltpu
from jax.experimental.pallas import tpu_sc as plsc

_B = 2048
_D_IN = 1024
_D_SAE = 8192
_K = _B * 32
_N = _B * _D_SAE

_NC = 2
_NS = 16
_NW = _NC * _NS
_PER_W = _N // _NW
_CHUNK = 16384
_NCHUNK = _PER_W // _CHUNK


def _encode_body(x_ref, w_ref, o_ref):
    acc = jnp.dot(x_ref[...], w_ref[...], preferred_element_type=jnp.float32)
    o_ref[...] = jnp.maximum(acc, 0.0)


def _encode(xc, w_enc):
    bn = 1024
    return pl.pallas_call(
        _encode_body,
        grid=(_D_SAE // bn,),
        in_specs=[
            pl.BlockSpec((_B, _D_IN), lambda n: (0, 0)),
            pl.BlockSpec((_D_IN, bn), lambda n: (0, n)),
        ],
        out_specs=pl.BlockSpec((_B, bn), lambda n: (0, n)),
        out_shape=jax.ShapeDtypeStruct((_B, _D_SAE), jnp.float32),
    )(xc, w_enc)


def _sc_hist1(flat):
    mesh = plsc.VectorSubcoreMesh(core_axis_name="c", subcore_axis_name="s")

    @functools.partial(
        pl.kernel,
        mesh=mesh,
        out_type=jax.ShapeDtypeStruct((_NW, 4096), jnp.int32),
        compiler_params=pltpu.CompilerParams(needs_layout_passes=False),
        scratch_types=[
            pltpu.VMEM((_CHUNK,), jnp.int32),
            pltpu.VMEM((_CHUNK,), jnp.int32),
            pltpu.VMEM((16 * 4096,), jnp.int32),
            pltpu.VMEM((4096,), jnp.int32),
            pltpu.SemaphoreType.DMA,
            pltpu.SemaphoreType.DMA,
        ],
    )
    def k(flat_hbm, out_hbm, buf0, buf1, hist, lred, sem0, sem1):
        sid = lax.axis_index("s")
        cid = lax.axis_index("c")
        wid = sid * _NC + cid
        lanes = lax.iota(jnp.int32, 16)
        lanes4k = lanes * 4096
        zeros16 = jnp.zeros((16,), jnp.int32)
        ones16 = jnp.ones((16,), jnp.int32)

        def zh(j, _):
            for u in range(8):
                hist[pl.ds(j * 128 + u * 16, 16)] = zeros16
            return 0

        lax.fori_loop(0, 512, zh, 0)

        base = wid * _PER_W

        def process(buf):
            def inner(i, _):
                for u in range(8):
                    bits = jnp.maximum(buf[pl.ds(i * 128 + u * 16, 16)], 0)
                    bkt = lax.shift_right_logical(bits, 19)
                    plsc.addupdate_scatter(hist, [bkt + lanes4k], ones16)
                return 0

            lax.fori_loop(0, _CHUNK // 128, inner, 0)

        pltpu.async_copy(flat_hbm.at[pl.ds(base, _CHUNK)], buf0, sem0)

        def chunk_body(g, _):
            off1 = base + (2 * g + 1) * _CHUNK
            pltpu.async_copy(flat_hbm.at[pl.ds(off1, _CHUNK)], buf1, sem1)
            pltpu.make_async_copy(flat_hbm.at[pl.ds(0, _CHUNK)], buf0, sem0).wait()
            process(buf0)
            off2 = jnp.minimum(base + (2 * g + 2) * _CHUNK, _N - _CHUNK)
            pltpu.async_copy(flat_hbm.at[pl.ds(off2, _CHUNK)], buf0, sem0)
            pltpu.make_async_copy(flat_hbm.at[pl.ds(0, _CHUNK)], buf1, sem1).wait()
            process(buf1)
            return 0

        lax.fori_loop(0, _NCHUNK // 2, chunk_body, 0)
        pltpu.make_async_copy(flat_hbm.at[pl.ds(0, _CHUNK)], buf0, sem0).wait()

        def red(j, _):
            acc = zeros16
            for lane in range(16):
                acc = acc + hist[pl.ds(lane * 4096 + j * 16, 16)]
            lred[pl.ds(j * 16, 16)] = acc
            return 0

        lax.fori_loop(0, 256, red, 0)
        pltpu.sync_copy(lred, out_hbm.at[wid])

    return k(flat)


def _scan1_body(h1_ref, o_ref):
    h = jnp.sum(h1_ref[...].astype(jnp.float32), axis=0)
    h2d = h.reshape(32, 128)
    li = lax.broadcasted_iota(jnp.int32, (128, 128), 0)
    lj = lax.broadcasted_iota(jnp.int32, (128, 128), 1)
    lmat = (li <= lj).astype(jnp.float32)
    rowcum = jnp.dot(h2d, lmat, preferred_element_type=jnp.float32,
                     precision=lax.Precision.HIGHEST)
    si = lax.broadcasted_iota(jnp.int32, (32, 32), 0)
    sj = lax.broadcasted_iota(jnp.int32, (32, 32), 1)
    smat = (sj < si).astype(jnp.float32)
    rowtot = rowcum[:, 127:128]
    rowoff = jnp.dot(smat, rowtot, preferred_element_type=jnp.float32,
                     precision=lax.Precision.HIGHEST)
    prefexc = rowcum + rowoff - h2d
    total = jnp.sum(h2d)
    cond = (prefexc <= (total - float(_K))).astype(jnp.float32)
    b1 = jnp.sum(cond).astype(jnp.int32) - 1
    b1f = jnp.sum(cond) - 1.0
    flat_idx = (lax.broadcasted_iota(jnp.int32, (32, 128), 0) * 128 +
                lax.broadcasted_iota(jnp.int32, (32, 128), 1)).astype(jnp.float32)
    sel = (flat_idx == b1f).astype(jnp.float32)
    prefinc_b1 = jnp.sum(sel * (prefexc + h2d))
    r = jnp.float32(_K) - (total - prefinc_b1)
    o_ref[0, :] = jnp.broadcast_to(b1, (128,))
    o_ref[1, :] = jnp.broadcast_to(r.astype(jnp.int32), (128,))


def _scan1(h1):
    return pl.pallas_call(
        _scan1_body,
        in_specs=[pl.BlockSpec((_NW, 4096), lambda: (0, 0))],
        out_specs=pl.BlockSpec((2, 128), lambda: (0, 0)),
        out_shape=jax.ShapeDtypeStruct((2, 128), jnp.int32),
    )(h1)


def _sc_hist2(flat, b1vec):
    mesh = plsc.VectorSubcoreMesh(core_axis_name="c", subcore_axis_name="s")

    @functools.partial(
        pl.kernel,
        mesh=mesh,
        out_type=jax.ShapeDtypeStruct((_NW, 65536), jnp.int32),
        compiler_params=pltpu.CompilerParams(needs_layout_passes=False),
        scratch_types=[
            pltpu.VMEM((_CHUNK,), jnp.int32),
            pltpu.VMEM((_CHUNK,), jnp.int32),
            pltpu.VMEM((65536,), jnp.int32),
            pltpu.VMEM((16,), jnp.int32),
            pltpu.SemaphoreType.DMA,
            pltpu.SemaphoreType.DMA,
        ],
    )
    def k(flat_hbm, b1_hbm, out_hbm, buf0, buf1, hist, b1buf, sem0, sem1):
        sid = lax.axis_index("s")
        cid = lax.axis_index("c")
        wid = sid * _NC + cid
        zeros16 = jnp.zeros((16,), jnp.int32)
        ones16 = jnp.ones((16,), jnp.int32)

        pltpu.sync_copy(b1_hbm, b1buf)
        b1v = b1buf[pl.ds(0, 16)]

        def zh(j, _):
            for u in range(8):
                hist[pl.ds(j * 128 + u * 16, 16)] = zeros16
            return 0

        lax.fori_loop(0, 512, zh, 0)

        base = wid * _PER_W

        def process(buf):
            def inner(i, _):
                for u in range(8):
                    bits = jnp.maximum(buf[pl.ds(i * 128 + u * 16, 16)], 0)
                    bkt = lax.shift_right_logical(bits, 19)
                    m = bkt == b1v
                    sub = lax.shift_right_logical(bits, 3) & 0xFFFF
                    plsc.addupdate_scatter(hist, [sub], ones16, mask=m)
                return 0

            lax.fori_loop(0, _CHUNK // 128, inner, 0)

        pltpu.async_copy(flat_hbm.at[pl.ds(base, _CHUNK)], buf0, sem0)

        def chunk_body(g, _):
            off1 = base + (2 * g + 1) * _CHUNK
            pltpu.async_copy(flat_hbm.at[pl.ds(off1, _CHUNK)], buf1, sem1)
            pltpu.make_async_copy(flat_hbm.at[pl.ds(0, _CHUNK)], buf0, sem0).wait()
            process(buf0)
            off2 = jnp.minimum(base + (2 * g + 2) * _CHUNK, _N - _CHUNK)
            pltpu.async_copy(flat_hbm.at[pl.ds(off2, _CHUNK)], buf0, sem0)
            pltpu.make_async_copy(flat_hbm.at[pl.ds(0, _CHUNK)], buf1, sem1).wait()
            process(buf1)
            return 0

        lax.fori_loop(0, _NCHUNK // 2, chunk_body, 0)
        pltpu.make_async_copy(flat_hbm.at[pl.ds(0, _CHUNK)], buf0, sem0).wait()
        pltpu.sync_copy(hist, out_hbm.at[wid])

    return k(flat, b1vec)


def _decode_body(h2_ref, binfo_ref, dense_ref, w_ref, bdec_ref,
                 recon_ref, sparse_ref, t_ref):
    j = pl.program_id(0)

    @pl.when(j == 0)
    def _():
        h = jnp.sum(h2_ref[...].astype(jnp.float32), axis=0)
        li = lax.broadcasted_iota(jnp.int32, (128, 128), 0)
        lj = lax.broadcasted_iota(jnp.int32, (128, 128), 1)
        lmat = (li <= lj).astype(jnp.float32)
        rowcum = jnp.dot(h, lmat, preferred_element_type=jnp.float32,
                         precision=lax.Precision.HIGHEST)
        si = lax.broadcasted_iota(jnp.int32, (512, 512), 0)
        sj = lax.broadcasted_iota(jnp.int32, (512, 512), 1)
        smat = (sj < si).astype(jnp.float32)
        rowtot = rowcum[:, 127:128]
        rowoff = jnp.dot(smat, rowtot, preferred_element_type=jnp.float32,
                         precision=lax.Precision.HIGHEST)
        prefexc = rowcum + rowoff - h
        total = jnp.sum(h)
        r = binfo_ref[0, 1].astype(jnp.float32)
        cond = (prefexc <= (total - r)).astype(jnp.float32)
        b2 = jnp.sum(cond).astype(jnp.int32) - 1
        b1 = binfo_ref[0, 0]
        t_ref[0, 0] = (b1 << 19) | (b2 << 3)

    tbits = t_ref[0, 0]
    d = dense_ref[...]
    dbits = lax.bitcast_convert_type(d, jnp.int32)
    s = jnp.where(dbits >= tbits, d, 0.0)
    sparse_ref[...] = s

    @pl.when(j == 0)
    def _():
        recon_ref[...] = jnp.broadcast_to(bdec_ref[...], recon_ref.shape)

    recon_ref[...] += jnp.dot(s, w_ref[...], preferred_element_type=jnp.float32)


def _decode(dense, w_dec, bdec2d, h2, binfo):
    bk = 512
    return pl.pallas_call(
        _decode_body,
        grid=(_D_SAE // bk,),
        in_specs=[
            pl.BlockSpec((_NW, 512, 128), lambda n: (0, 0, 0)),
            pl.BlockSpec(memory_space=pltpu.SMEM),
            pl.BlockSpec((_B, bk), lambda n: (0, n)),
            pl.BlockSpec((bk, _D_IN), lambda n: (n, 0)),
            pl.BlockSpec((1, _D_IN), lambda n: (0, 0)),
        ],
        out_specs=[
            pl.BlockSpec((_B, _D_IN), lambda n: (0, 0)),
            pl.BlockSpec((_B, bk), lambda n: (0, n)),
        ],
        out_shape=[
            jax.ShapeDtypeStruct((_B, _D_IN), jnp.float32),
            jax.ShapeDtypeStruct((_B, _D_SAE), jnp.float32),
        ],
        scratch_shapes=[pltpu.SMEM((1, 1), jnp.int32)],
    )(h2, binfo, dense, w_dec, bdec2d)


def kernel(x, W_enc, W_dec, b_dec):
    xc = x - b_dec
    dense = _encode(xc, W_enc)
    flat = lax.bitcast_convert_type(dense, jnp.int32).reshape(-1)
    h1 = _sc_hist1(flat)
    info = _scan1(h1)
    b1vec = info[0, :16]
    binfo = info[:, :1].reshape(1, 2)
    h2 = _sc_hist2(flat, b1vec)
    x_recon, sparse = _decode(dense, W_dec, b_dec.reshape(1, _D_IN),
                              h2.reshape(_NW, 512, 128), binfo)
    return x_recon, sparse
